# baseline (speedup 1.0000x reference)
import jax
import jax.numpy as jnp
from jax.experimental import pallas as pl
from jax.experimental.pallas import tpu as pltpu

_N = 8192
_D = 128
_BI = 4096
_BK = 256
_MH = 136


def _body(adj_ref, x_ref, u_ref, out_ref, xh_ref, acc_ref):
    i = pl.program_id(0)
    k = pl.program_id(1)
    nk = pl.num_programs(1)

    @pl.when((i == 0) & (k == 0))
    def _setup_x():
        xh_ref[0:_D, :] = x_ref[...].T.astype(jnp.bfloat16)
        xh_ref[_D:_D + 1, :] = jnp.ones((1, _N), jnp.bfloat16)
        xh_ref[_D + 1:, :] = jnp.zeros((_MH - _D - 1, _N), jnp.bfloat16)

    @pl.when(k == 0)
    def _init():
        acc_ref[...] = jnp.zeros_like(acc_ref)

    a = adj_ref[...].astype(jnp.bfloat16)
    xh = xh_ref[:, pl.ds(k * _BK, _BK)]
    dims = (((1,), (0,)), ((), ()))
    hi = jax.lax.dot_general(xh, a, dims, preferred_element_type=jnp.float32)
    acc_ref[...] += hi

    @pl.when(k == nk - 1)
    def _epilogue():
        agg = acc_ref[0:_D, :].astype(jnp.bfloat16)
        deg = acc_ref[_D:_D + 1, :]
        u16 = u_ref[...].astype(jnp.bfloat16)
        y = jnp.dot(u16, agg, preferred_element_type=jnp.float32)
        y = jnp.maximum(y / deg, 0.0)
        out_ref[...] = y.T


def kernel(x, adj_mat, U):
    return pl.pallas_call(
        _body,
        grid=(_N // _BI, _N // _BK),
        in_specs=[
            pl.BlockSpec((_BK, _BI), lambda i, k: (k, i)),
            pl.BlockSpec((_N, _D), lambda i, k: (0, 0)),
            pl.BlockSpec((_D, _D), lambda i, k: (0, 0)),
        ],
        out_specs=pl.BlockSpec((_BI, _D), lambda i, k: (i, 0)),
        out_shape=jax.ShapeDtypeStruct((_N, _D), jnp.float32),
        scratch_shapes=[
            pltpu.VMEM((_MH, _N), jnp.bfloat16),
            pltpu.VMEM((_MH, _BI), jnp.float32),
        ],
        compiler_params=pltpu.CompilerParams(
            dimension_semantics=("arbitrary", "arbitrary")),
    )(adj_mat, x, U)


# speedup vs baseline: 1.1542x; 1.1542x over previous
import jax
import jax.numpy as jnp
from jax.experimental import pallas as pl
from jax.experimental.pallas import tpu as pltpu

_N = 8192
_D = 128
_BI = 4096
_BK = 512
_MH = 136


def _body(adj_ref, x_ref, u_ref, out_ref, xh_ref, acc_ref):
    i = pl.program_id(0)
    k = pl.program_id(1)
    nk = pl.num_programs(1)

    @pl.when((i == 0) & (k == 0))
    def _setup_x():
        xh_ref[0:_D, :] = x_ref[...].T.astype(jnp.bfloat16)
        xh_ref[_D:_D + 1, :] = jnp.ones((1, _N), jnp.bfloat16)
        xh_ref[_D + 1:, :] = jnp.zeros((_MH - _D - 1, _N), jnp.bfloat16)

    @pl.when(k == 0)
    def _init():
        acc_ref[...] = jnp.zeros_like(acc_ref)

    a = adj_ref[...].astype(jnp.bfloat16)
    xh = xh_ref[:, pl.ds(k * _BK, _BK)]
    dims = (((1,), (0,)), ((), ()))
    hi = jax.lax.dot_general(xh, a, dims, preferred_element_type=jnp.float32)
    acc_ref[...] += hi

    @pl.when(k == nk - 1)
    def _epilogue():
        agg = acc_ref[0:_D, :].astype(jnp.bfloat16)
        deg = acc_ref[_D:_D + 1, :]
        u16 = u_ref[...].astype(jnp.bfloat16)
        y = jnp.dot(u16, agg, preferred_element_type=jnp.float32)
        y = jnp.maximum(y / deg, 0.0)
        out_ref[...] = y.T


def kernel(x, adj_mat, U):
    return pl.pallas_call(
        _body,
        grid=(_N // _BI, _N // _BK),
        in_specs=[
            pl.BlockSpec((_BK, _BI), lambda i, k: (k, i)),
            pl.BlockSpec((_N, _D), lambda i, k: (0, 0)),
            pl.BlockSpec((_D, _D), lambda i, k: (0, 0)),
        ],
        out_specs=pl.BlockSpec((_BI, _D), lambda i, k: (i, 0)),
        out_shape=jax.ShapeDtypeStruct((_N, _D), jnp.float32),
        scratch_shapes=[
            pltpu.VMEM((_MH, _N), jnp.bfloat16),
            pltpu.VMEM((_MH, _BI), jnp.float32),
        ],
        compiler_params=pltpu.CompilerParams(
            dimension_semantics=("arbitrary", "arbitrary")),
    )(adj_mat, x, U)
